# Initial kernel scaffold; baseline (speedup 1.0000x reference)
import jax
import jax.numpy as jnp
from jax.experimental import pallas as pl


def kernel(x, loc_orig, idx_agg, agg_weight, map_h, map_w, Wq, bq, Wk, bk, Wv, bv, Wo, bo):
    raise NotImplementedError("write your pallas kernel here")


# speedup vs baseline: 2.8660x; 2.8660x over previous
import functools

import jax
import jax.numpy as jnp
from jax.experimental import pallas as pl
from jax.experimental.pallas import tpu as pltpu

B, N, N0, C = 4, 784, 3136, 384
HEADS, HD = 12, 32
H = W = 56
KW = 7
NW_SIDE = 8
NWIN = 64
K = 49
SCALE = float(HD) ** -0.5

_pc = functools.partial(pl.pallas_call, interpret=False)


def _stage_a_xla(x, loc_orig, idx_agg, agg_weight):
    loc = jnp.clip(loc_orig, -1.0, 1.0)
    px = 0.5 * (loc[..., 0] + 1.0) * W - 0.5
    py = 0.5 * (loc[..., 1] + 1.0) * H - 0.5
    ix = jnp.clip(jnp.round(px).astype(jnp.int32), 0, W - 1)
    iy = jnp.clip(jnp.round(py).astype(jnp.int32), 0, H - 1)
    win = (iy // KW) * NW_SIDE + (ix // KW)
    pos = (iy % KW) * KW + (ix % KW)
    idx_perm = win * K + pos
    b_idx = jnp.arange(B)[:, None]
    tok = x[b_idx, idx_agg]
    acc = jnp.zeros((B, N0, C), x.dtype).at[b_idx, idx_perm].add(tok)
    cnt = jnp.zeros((B, N0), x.dtype).at[b_idx, idx_perm].add(1.0)
    aw = agg_weight[..., 0]
    num = jnp.zeros((B, N, NWIN), x.dtype).at[b_idx, idx_agg, win].add(aw)
    idx_win = jnp.argmax(num, axis=-1).astype(jnp.int32)
    return acc, cnt, idx_win


def _proj_body(acc_ref, cnt_ref, x_ref, wqt_ref, bq_ref, wkt_ref, bk_ref,
               wvt_ref, bv_ref, q_ref, k_ref, v_ref):
    inv = 1.0 / (cnt_ref[0] + 1e-6)
    xm = acc_ref[0] * inv
    q = jnp.dot(x_ref[0], wqt_ref[...], preferred_element_type=jnp.float32)
    q_ref[0] = (q + bq_ref[...]) * SCALE
    k = jnp.dot(xm, wkt_ref[...], preferred_element_type=jnp.float32)
    k_ref[0] = k + bk_ref[...]
    v = jnp.dot(xm, wvt_ref[...], preferred_element_type=jnp.float32)
    v_ref[0] = v + bv_ref[...]


def _projections(acc, cnt, x, WqT, bq, WkT, bk, WvT, bv):
    full = lambda *shape: pl.BlockSpec(shape, lambda b: (0,) * len(shape))
    grid_spec = pl.GridSpec(
        grid=(B,),
        in_specs=[
            pl.BlockSpec((1, N0, C), lambda b: (b, 0, 0)),
            pl.BlockSpec((1, N0, 1), lambda b: (b, 0, 0)),
            pl.BlockSpec((1, N, C), lambda b: (b, 0, 0)),
            full(C, C), full(1, C), full(C, C), full(1, C), full(C, C), full(1, C),
        ],
        out_specs=[
            pl.BlockSpec((1, N, C), lambda b: (b, 0, 0)),
            pl.BlockSpec((1, N0, C), lambda b: (b, 0, 0)),
            pl.BlockSpec((1, N0, C), lambda b: (b, 0, 0)),
        ],
    )
    return _pc(
        _proj_body,
        grid_spec=grid_spec,
        out_shape=[
            jax.ShapeDtypeStruct((B, N, C), jnp.float32),
            jax.ShapeDtypeStruct((B, N0, C), jnp.float32),
            jax.ShapeDtypeStruct((B, N0, C), jnp.float32),
        ],
    )(acc, cnt.reshape(B, N0, 1), x, WqT, bq.reshape(1, C),
      WkT, bk.reshape(1, C), WvT, bv.reshape(1, C))


def _attn_body(q_ref, k_ref, v_ref, iw_ref, wot_ref, bo_ref, o_ref):
    colwin = jax.lax.broadcasted_iota(jnp.int32, (1, N0), 1) // K
    bias = jnp.where(iw_ref[0] == colwin, 0.0, -1e30)
    outs = []
    for h in range(HEADS):
        sl = slice(h * HD, (h + 1) * HD)
        qh = q_ref[0, :, sl]
        kh = k_ref[0, :, sl]
        s = jax.lax.dot_general(qh, kh, (((1,), (1,)), ((), ())),
                                preferred_element_type=jnp.float32) + bias
        m = jnp.max(s, axis=1, keepdims=True)
        p = jnp.exp(s - m)
        p = p / jnp.sum(p, axis=1, keepdims=True)
        outs.append(jnp.dot(p, v_ref[0, :, sl],
                            preferred_element_type=jnp.float32))
    o = jnp.concatenate(outs, axis=1)
    o_ref[0] = jnp.dot(o, wot_ref[...],
                       preferred_element_type=jnp.float32) + bo_ref[...]


def _attention(q, k, v, idx_win, WoT, bo):
    full = lambda *shape: pl.BlockSpec(shape, lambda b: (0,) * len(shape))
    grid_spec = pl.GridSpec(
        grid=(B,),
        in_specs=[
            pl.BlockSpec((1, N, C), lambda b: (b, 0, 0)),
            pl.BlockSpec((1, N0, C), lambda b: (b, 0, 0)),
            pl.BlockSpec((1, N0, C), lambda b: (b, 0, 0)),
            pl.BlockSpec((1, N, 1), lambda b: (b, 0, 0)),
            full(C, C), full(1, C),
        ],
        out_specs=pl.BlockSpec((1, N, C), lambda b: (b, 0, 0)),
    )
    return _pc(
        _attn_body,
        grid_spec=grid_spec,
        out_shape=jax.ShapeDtypeStruct((B, N, C), jnp.float32),
    )(q, k, v, idx_win.reshape(B, N, 1), WoT, bo.reshape(1, C))


def kernel(x, loc_orig, idx_agg, agg_weight, map_h, map_w,
           Wq, bq, Wk, bk, Wv, bv, Wo, bo):
    acc, cnt, idx_win = _stage_a_xla(x, loc_orig, idx_agg, agg_weight)
    q, k, v = _projections(acc, cnt, x, Wq.T, bq, Wk.T, bk, Wv.T, bv)
    return _attention(q, k, v, idx_win, Wo.T, bo)
